# baseline (speedup 1.0000x reference)
import functools

import jax
import jax.numpy as jnp
from jax import lax
from jax.experimental import pallas as pl
from jax.experimental.pallas import tpu as pltpu
from jax.experimental.pallas import tpu_sc as plsc

N = 100000
T = 10
E = 1600000

NPAD = 100352
ROWS_PER_SUB = N // 16

PR = NPAD * 16 // 128
PRN = N * 16 // 128
PBN = 256
GRID = PR // PBN

EROWS = 12672
EPAD = EROWS * 128
NZSPREAD = 352
SUB_ROWS = EROWS // 16

_f32 = jnp.float32


@functools.cache
def _mesh():
    return plsc.VectorSubcoreMesh(core_axis_name="c", subcore_axis_name="s")


def _sc_kernel(**kw):
    def deco(f):
        @functools.wraps(f)
        def call(*args):
            return pl.kernel(
                f, mesh=_mesh(),
                compiler_params=pltpu.CompilerParams(use_tc_tiling_on_sc=False),
                **kw)(*args)
        return call
    return deco


@_sc_kernel(
    out_type=[jax.ShapeDtypeStruct((NPAD, 16), _f32),
              jax.ShapeDtypeStruct((NPAD, 16), _f32)],
    scratch_types=[
        pltpu.VMEM((12, 128), jnp.int32),
        pltpu.VMEM((12, 128), jnp.int32),
        pltpu.VMEM((6, 128, 16), _f32),
        pltpu.VMEM((6, 128, 16), _f32),
        pltpu.VMEM_SHARED((N, 16), _f32),
        pltpu.SemaphoreType.DMA,
        pltpu.SemaphoreType.DMA,
    ],
)
def _sc_seg(src_hbm, dst_hbm, zeros_hbm, glo_hbm, ghi_hbm, out0, out1,
            sidx, didx, rowsa, rowsb, acc, sema, semb):
    c = lax.axis_index("c")
    s = lax.axis_index("s")

    base = s * ROWS_PER_SUB
    pltpu.sync_copy(zeros_hbm, acc.at[pl.ds(base, ROWS_PER_SUB)])
    plsc.subcore_barrier()

    def _edge_loop(tbl):
        r0 = s * SUB_ROWS

        def _group(gi, _):
            row = r0 + gi * 12
            pltpu.sync_copy(src_hbm.at[pl.ds(row, 12), :], sidx)
            pltpu.sync_copy(dst_hbm.at[pl.ds(row, 12), :], didx)
            cpsa = [pltpu.async_copy(tbl.at[sidx.at[j]], rowsa.at[j], sema)
                    for j in range(6)]
            cpsb = [pltpu.async_copy(tbl.at[sidx.at[6 + j]], rowsb.at[j],
                                     semb) for j in range(6)]
            for cp in cpsa:
                cp.wait()
            for j in range(6):
                pltpu.sync_copy(rowsa.at[j], acc.at[didx.at[j]], add=True)
            for cp in cpsb:
                cp.wait()
            for j in range(6):
                pltpu.sync_copy(rowsb.at[j], acc.at[didx.at[6 + j]], add=True)
            return 0

        lax.fori_loop(0, SUB_ROWS // 12, _group, 0)

    @pl.when(c == 0)
    def _():
        _edge_loop(glo_hbm)

    @pl.when(c == 1)
    def _():
        _edge_loop(ghi_hbm)

    plsc.subcore_barrier()

    sl = pl.ds(base, ROWS_PER_SUB)

    @pl.when(c == 0)
    def _():
        pltpu.sync_copy(acc.at[sl], out0.at[sl])

    @pl.when(c == 1)
    def _():
        pltpu.sync_copy(acc.at[sl], out1.at[sl])


def _seg(src2d, dst2d, zeros_hbm, glo_p, ghi_p):
    s0, s1 = _sc_seg(src2d, dst2d, zeros_hbm,
                     glo_p.reshape(NPAD, 16), ghi_p.reshape(NPAD, 16))
    return s0.reshape(PR, 128), s1.reshape(PR, 128)


def _p_spec():
    return pl.BlockSpec((PBN, 128), lambda i: (i, 0))


def _w_spec():
    return pl.BlockSpec((128, 128), lambda i: (0, 0))


def _c_spec():
    return pl.BlockSpec((8, 128), lambda i: (0, 0))


def _pmask():
    row = pl.program_id(0) * PBN + lax.broadcasted_iota(jnp.int32, (PBN, 1), 0)
    return row < PRN


def _mm(a, w):
    return jnp.dot(a, w, preferred_element_type=_f32)


def _tc_prep(dg0_p):
    def k(a, o):
        o[:, :] = lax.rsqrt(a[:, :] + 1.0)

    return pl.pallas_call(
        k,
        grid=(GRID,),
        in_specs=[_p_spec()],
        out_specs=_p_spec(),
        out_shape=jax.ShapeDtypeStruct((PR, 128), _f32),
    )(dg0_p)


def _tc_init(y0p, xp, dvp, wb, cst):
    def k(y0, xc, dv, w, cs, cs2, glo, ghi):
        hlo = jnp.maximum(y0[:, :] * cs[0, :] + cs[2, :], 0.0)
        hhi = jnp.maximum(y0[:, :] * cs[1, :] + cs[3, :], 0.0)
        hlo2 = jnp.maximum(
            _mm(hlo, w[0, 0]) + _mm(hhi, w[1, 0]) + cs[4, :], 0.0)
        hhi2 = jnp.maximum(
            _mm(hlo, w[0, 1]) + _mm(hhi, w[1, 1]) + cs[5, :], 0.0)
        m = _pmask()
        dm = jnp.where(m, dv[:, :], 0.0)
        glo[:, :] = dm * (_mm(hlo2, w[2, 0]) + _mm(hhi2, w[3, 0])
                          + xc[:, :] * cs[6, :] + cs2[0, :])
        ghi[:, :] = dm * (_mm(hlo2, w[2, 1]) + _mm(hhi2, w[3, 1])
                          + xc[:, :] * cs[7, :] + cs2[1, :])

    return pl.pallas_call(
        k,
        grid=(GRID,),
        in_specs=[_p_spec(), _p_spec(), _p_spec(),
                  pl.BlockSpec((4, 2, 128, 128), lambda i: (0, 0, 0, 0)),
                  _c_spec(), _c_spec()],
        out_specs=[_p_spec(), _p_spec()],
        out_shape=[jax.ShapeDtypeStruct((PR, 128), _f32),
                   jax.ShapeDtypeStruct((PR, 128), _f32)],
    )(y0p, xp, dvp, wb, cst[0], cst[1])


def _tc_mid(s0, s1, glo, ghi, dvp, wb, cst):
    def k(a0, a1, g0, g1, dv, w, cs, olo, ohi):
        d = dv[:, :]
        hlo = jnp.maximum(d * (a0[:, :] + g0[:, :]) + cs[0, :], 0.0)
        hhi = jnp.maximum(d * (a1[:, :] + g1[:, :]) + cs[1, :], 0.0)
        m = _pmask()
        dm = jnp.where(m, d, 0.0)
        olo[:, :] = dm * (_mm(hlo, w[0, 0]) + _mm(hhi, w[1, 0]))
        ohi[:, :] = dm * (_mm(hlo, w[0, 1]) + _mm(hhi, w[1, 1]))

    return pl.pallas_call(
        k,
        grid=(GRID,),
        in_specs=[_p_spec()] * 5
        + [pl.BlockSpec((2, 2, 128, 128), lambda i: (0, 0, 0, 0)), _c_spec()],
        out_specs=[_p_spec(), _p_spec()],
        out_shape=[jax.ShapeDtypeStruct((PR, 128), _f32),
                   jax.ShapeDtypeStruct((PR, 128), _f32)],
    )(s0, s1, glo, ghi, dvp, wb, cst)


def _tc_tail(s0, s1, glo, ghi, dvp, xp, wb, cst, cst2):
    def k(a0, a1, g0, g1, dv, xc, w, cs, cs2, yp, olo, ohi):
        d = dv[:, :]
        hlo = jnp.maximum(d * (a0[:, :] + g0[:, :]) + cs[0, :], 0.0)
        hhi = jnp.maximum(d * (a1[:, :] + g1[:, :]) + cs[1, :], 0.0)
        qlo = jnp.maximum(_mm(hlo, w[0]) + _mm(hhi, w[1]) + cs[2, :], 0.0)
        qhi = jnp.maximum(_mm(hlo, w[2]) + _mm(hhi, w[3]) + cs[3, :], 0.0)
        yp[:, :] = _mm(qlo, w[4]) + _mm(qhi, w[5]) + cs[4, :]
        m = _pmask()
        dm = jnp.where(m, d, 0.0)
        olo[:, :] = dm * (_mm(hlo, w[6]) + _mm(hhi, w[7])
                          + xc[:, :] * cs[6, :] + cs2[0, :])
        ohi[:, :] = dm * (_mm(hlo, w[8]) + _mm(hhi, w[9])
                          + xc[:, :] * cs[7, :] + cs2[1, :])

    return pl.pallas_call(
        k,
        grid=(GRID,),
        in_specs=[_p_spec()] * 6
        + [pl.BlockSpec((10, 128, 128), lambda i: (0, 0, 0)),
           _c_spec(), _c_spec()],
        out_specs=[_p_spec()] * 3,
        out_shape=[jax.ShapeDtypeStruct((PR, 128), _f32)] * 3,
    )(s0, s1, glo, ghi, dvp, xp, wb, cst, cst2)


def kernel(x, t, y, fc1_W, fc1_b, fc2_W, fc2_b, conv1_W, conv1_b,
           conv2_W, conv2_b, conv3_W, conv3_b, fc3_W, fc3_b,
           fc4_W, fc4_b, edge_index):
    pad_i = jnp.arange(EPAD - E, dtype=jnp.int32)
    src2d = jnp.concatenate([edge_index[0], N + pad_i % NZSPREAD]
                            ).reshape(EROWS, 128)
    dst2d = jnp.concatenate([edge_index[1], pad_i % N]).reshape(EROWS, 128)

    zeros_hbm = jnp.zeros((ROWS_PER_SUB, 16), _f32)
    ones_tbl = jnp.zeros((NPAD, 16), _f32).at[:N].set(1.0)

    xp = jnp.zeros((NPAD, 16), _f32).at[:N].set(
        jnp.broadcast_to(x, (N, 16))).reshape(PR, 128)
    y0p = jnp.zeros((NPAD, 16), _f32).at[:N].set(
        jnp.broadcast_to(y[0][:, None], (N, 16))).reshape(PR, 128)

    eye8 = jnp.eye(8, dtype=_f32)

    def bd(m):
        return jnp.kron(eye8, m)

    def bd4(w):
        return jnp.stack([
            jnp.stack([bd(w[:16, :16]), bd(w[:16, 16:])]),
            jnp.stack([bd(w[16:, :16]), bd(w[16:, 16:])])])

    def t16(v):
        return jnp.tile(v, 8)

    def pad32(v, n):
        return jnp.zeros((32,), _f32).at[:n].set(v)

    def rows8(*rows):
        c = jnp.zeros((8, 128), _f32)
        for i, r in enumerate(rows):
            c = c.at[i].set(r)
        return c

    fc2p = jnp.zeros((32, 32), _f32).at[:, :26].set(fc2_W)
    b2p = pad32(fc2_b, 26)
    w1ap = jnp.zeros((32, 32), _f32).at[:26].set(conv1_W[:26])
    wx1 = conv1_W[26:29].sum(0)
    tbt = t[:, None] * conv1_W[29:32].sum(0)[None, :]
    w3p = jnp.zeros((32, 32), _f32).at[:, :26].set(conv3_W)
    b3p = pad32(conv3_b, 26)
    fc3p = jnp.zeros((32, 32), _f32).at[:26].set(fc3_W)
    fc4p = jnp.zeros((32, 16), _f32).at[:, 0].set(fc4_W[:, 0])

    fc1r = jnp.broadcast_to(fc1_W[0], (32,))

    wb_init = jnp.concatenate([bd4(fc2p), bd4(w1ap)], 0)
    cst_init = (
        rows8(t16(fc1r[:16]), t16(fc1r[16:]), t16(fc1_b[:16]),
              t16(fc1_b[16:]), t16(b2p[:16]), t16(b2p[16:]),
              t16(wx1[:16]), t16(wx1[16:])),
        rows8(t16(tbt[1, :16]), t16(tbt[1, 16:])),
    )
    wb_c2 = bd4(conv2_W)
    cst_c1 = rows8(t16(conv1_b[:16]), t16(conv1_b[16:]))
    wb_c3 = bd4(w3p)
    cst_c2 = rows8(t16(conv2_b[:16]), t16(conv2_b[16:]))
    f3 = bd4(fc3p)
    w1a4 = bd4(w1ap)
    wb_tail = jnp.stack([
        f3[0, 0], f3[1, 0], f3[0, 1], f3[1, 1],
        bd(fc4p[:16]), bd(fc4p[16:]),
        w1a4[0, 0], w1a4[1, 0], w1a4[0, 1], w1a4[1, 1]])
    cst_tail = rows8(t16(b3p[:16]), t16(b3p[16:]), t16(fc3_b[:16]),
                     t16(fc3_b[16:]), jnp.full((128,), fc4_b[0], _f32),
                     jnp.zeros((128,), _f32), t16(wx1[:16]), t16(wx1[16:]))

    dg0, _ = _sc_seg(src2d, dst2d, zeros_hbm, ones_tbl, ones_tbl)
    dvp = _tc_prep(dg0.reshape(PR, 128))

    glo, ghi = _tc_init(y0p, xp, dvp, wb_init, cst_init)
    yrows = [y[0]]
    for i in range(1, T):
        s0, s1 = _seg(src2d, dst2d, zeros_hbm, glo, ghi)
        glo, ghi = _tc_mid(s0, s1, glo, ghi, dvp, wb_c2, cst_c1)
        s0, s1 = _seg(src2d, dst2d, zeros_hbm, glo, ghi)
        glo, ghi = _tc_mid(s0, s1, glo, ghi, dvp, wb_c3, cst_c2)
        s0, s1 = _seg(src2d, dst2d, zeros_hbm, glo, ghi)
        tb_next = tbt[i + 1] if i + 1 < T else jnp.zeros((32,), _f32)
        cst2 = rows8(t16(tb_next[:16]), t16(tb_next[16:]))
        yp, glo, ghi = _tc_tail(s0, s1, glo, ghi, dvp, xp,
                                wb_tail, cst_tail, cst2)
        yrows.append(yp.reshape(NPAD, 16)[:N, 0])

    res = jnp.stack(yrows, 0)
    return res.reshape(-1)


# speedup vs baseline: 1.0327x; 1.0327x over previous
import functools

import jax
import jax.numpy as jnp
from jax import lax
from jax.experimental import pallas as pl
from jax.experimental.pallas import tpu as pltpu
from jax.experimental.pallas import tpu_sc as plsc

N = 100000
T = 10
E = 1600000

NPAD = 100352
ROWS_PER_SUB = N // 16

PR = NPAD * 16 // 128
PRN = N * 16 // 128
PBN = 448
GRID = PR // PBN

EROWS = 12672
EPAD = EROWS * 128
NZSPREAD = 352
SUB_ROWS = EROWS // 16

_f32 = jnp.float32


@functools.cache
def _mesh():
    return plsc.VectorSubcoreMesh(core_axis_name="c", subcore_axis_name="s")


def _sc_kernel(**kw):
    def deco(f):
        @functools.wraps(f)
        def call(*args):
            return pl.kernel(
                f, mesh=_mesh(),
                compiler_params=pltpu.CompilerParams(use_tc_tiling_on_sc=False),
                **kw)(*args)
        return call
    return deco


@_sc_kernel(
    out_type=[jax.ShapeDtypeStruct((NPAD, 16), _f32),
              jax.ShapeDtypeStruct((NPAD, 16), _f32)],
    scratch_types=[
        pltpu.VMEM((12, 128), jnp.int32),
        pltpu.VMEM((12, 128), jnp.int32),
        pltpu.VMEM((6, 128, 16), _f32),
        pltpu.VMEM((6, 128, 16), _f32),
        pltpu.VMEM_SHARED((N, 16), _f32),
        pltpu.SemaphoreType.DMA,
        pltpu.SemaphoreType.DMA,
    ],
)
def _sc_seg(src_hbm, dst_hbm, zeros_hbm, glo_hbm, ghi_hbm, out0, out1,
            sidx, didx, rowsa, rowsb, acc, sema, semb):
    c = lax.axis_index("c")
    s = lax.axis_index("s")

    base = s * ROWS_PER_SUB
    pltpu.sync_copy(zeros_hbm, acc.at[pl.ds(base, ROWS_PER_SUB)])
    plsc.subcore_barrier()

    def _edge_loop(tbl):
        r0 = s * SUB_ROWS

        def _group(gi, _):
            row = r0 + gi * 12
            pltpu.sync_copy(src_hbm.at[pl.ds(row, 12), :], sidx)
            pltpu.sync_copy(dst_hbm.at[pl.ds(row, 12), :], didx)
            cpsa = [pltpu.async_copy(tbl.at[sidx.at[j]], rowsa.at[j], sema)
                    for j in range(6)]
            cpsb = [pltpu.async_copy(tbl.at[sidx.at[6 + j]], rowsb.at[j],
                                     semb) for j in range(6)]
            for cp in cpsa:
                cp.wait()
            for j in range(6):
                pltpu.sync_copy(rowsa.at[j], acc.at[didx.at[j]], add=True)
            for cp in cpsb:
                cp.wait()
            for j in range(6):
                pltpu.sync_copy(rowsb.at[j], acc.at[didx.at[6 + j]], add=True)
            return 0

        lax.fori_loop(0, SUB_ROWS // 12, _group, 0)

    @pl.when(c == 0)
    def _():
        _edge_loop(glo_hbm)

    @pl.when(c == 1)
    def _():
        _edge_loop(ghi_hbm)

    plsc.subcore_barrier()

    sl = pl.ds(base, ROWS_PER_SUB)

    @pl.when(c == 0)
    def _():
        pltpu.sync_copy(acc.at[sl], out0.at[sl])

    @pl.when(c == 1)
    def _():
        pltpu.sync_copy(acc.at[sl], out1.at[sl])


def _seg(src2d, dst2d, zeros_hbm, glo_p, ghi_p):
    s0, s1 = _sc_seg(src2d, dst2d, zeros_hbm,
                     glo_p.reshape(NPAD, 16), ghi_p.reshape(NPAD, 16))
    return s0.reshape(PR, 128), s1.reshape(PR, 128)


def _p_spec():
    return pl.BlockSpec((PBN, 128), lambda i: (i, 0))


def _w_spec():
    return pl.BlockSpec((128, 128), lambda i: (0, 0))


def _c_spec():
    return pl.BlockSpec((8, 128), lambda i: (0, 0))


def _pmask():
    row = pl.program_id(0) * PBN + lax.broadcasted_iota(jnp.int32, (PBN, 1), 0)
    return row < PRN


def _mm(a, w):
    return jnp.dot(a, w, preferred_element_type=_f32)


def _tc_prep(dg0_p):
    def k(a, o):
        o[:, :] = lax.rsqrt(a[:, :] + 1.0)

    return pl.pallas_call(
        k,
        grid=(GRID,),
        in_specs=[_p_spec()],
        out_specs=_p_spec(),
        out_shape=jax.ShapeDtypeStruct((PR, 128), _f32),
    )(dg0_p)


def _tc_init(y0p, xp, dvp, wb, cst):
    def k(y0, xc, dv, w, cs, cs2, glo, ghi):
        hlo = jnp.maximum(y0[:, :] * cs[0, :] + cs[2, :], 0.0)
        hhi = jnp.maximum(y0[:, :] * cs[1, :] + cs[3, :], 0.0)
        hlo2 = jnp.maximum(
            _mm(hlo, w[0, 0]) + _mm(hhi, w[1, 0]) + cs[4, :], 0.0)
        hhi2 = jnp.maximum(
            _mm(hlo, w[0, 1]) + _mm(hhi, w[1, 1]) + cs[5, :], 0.0)
        m = _pmask()
        dm = jnp.where(m, dv[:, :], 0.0)
        glo[:, :] = dm * (_mm(hlo2, w[2, 0]) + _mm(hhi2, w[3, 0])
                          + xc[:, :] * cs[6, :] + cs2[0, :])
        ghi[:, :] = dm * (_mm(hlo2, w[2, 1]) + _mm(hhi2, w[3, 1])
                          + xc[:, :] * cs[7, :] + cs2[1, :])

    return pl.pallas_call(
        k,
        grid=(GRID,),
        in_specs=[_p_spec(), _p_spec(), _p_spec(),
                  pl.BlockSpec((4, 2, 128, 128), lambda i: (0, 0, 0, 0)),
                  _c_spec(), _c_spec()],
        out_specs=[_p_spec(), _p_spec()],
        out_shape=[jax.ShapeDtypeStruct((PR, 128), _f32),
                   jax.ShapeDtypeStruct((PR, 128), _f32)],
    )(y0p, xp, dvp, wb, cst[0], cst[1])


def _tc_mid(s0, s1, glo, ghi, dvp, wb, cst):
    def k(a0, a1, g0, g1, dv, w, cs, olo, ohi):
        d = dv[:, :]
        hlo = jnp.maximum(d * (a0[:, :] + g0[:, :]) + cs[0, :], 0.0)
        hhi = jnp.maximum(d * (a1[:, :] + g1[:, :]) + cs[1, :], 0.0)
        m = _pmask()
        dm = jnp.where(m, d, 0.0)
        olo[:, :] = dm * (_mm(hlo, w[0, 0]) + _mm(hhi, w[1, 0]))
        ohi[:, :] = dm * (_mm(hlo, w[0, 1]) + _mm(hhi, w[1, 1]))

    return pl.pallas_call(
        k,
        grid=(GRID,),
        in_specs=[_p_spec()] * 5
        + [pl.BlockSpec((2, 2, 128, 128), lambda i: (0, 0, 0, 0)), _c_spec()],
        out_specs=[_p_spec(), _p_spec()],
        out_shape=[jax.ShapeDtypeStruct((PR, 128), _f32),
                   jax.ShapeDtypeStruct((PR, 128), _f32)],
    )(s0, s1, glo, ghi, dvp, wb, cst)


def _tc_tail(s0, s1, glo, ghi, dvp, xp, wb, cst, cst2):
    def k(a0, a1, g0, g1, dv, xc, w, cs, cs2, yp, olo, ohi):
        d = dv[:, :]
        hlo = jnp.maximum(d * (a0[:, :] + g0[:, :]) + cs[0, :], 0.0)
        hhi = jnp.maximum(d * (a1[:, :] + g1[:, :]) + cs[1, :], 0.0)
        qlo = jnp.maximum(_mm(hlo, w[0]) + _mm(hhi, w[1]) + cs[2, :], 0.0)
        qhi = jnp.maximum(_mm(hlo, w[2]) + _mm(hhi, w[3]) + cs[3, :], 0.0)
        yp[:, :] = _mm(qlo, w[4]) + _mm(qhi, w[5]) + cs[4, :]
        m = _pmask()
        dm = jnp.where(m, d, 0.0)
        olo[:, :] = dm * (_mm(hlo, w[6]) + _mm(hhi, w[7])
                          + xc[:, :] * cs[6, :] + cs2[0, :])
        ohi[:, :] = dm * (_mm(hlo, w[8]) + _mm(hhi, w[9])
                          + xc[:, :] * cs[7, :] + cs2[1, :])

    return pl.pallas_call(
        k,
        grid=(GRID,),
        in_specs=[_p_spec()] * 6
        + [pl.BlockSpec((10, 128, 128), lambda i: (0, 0, 0)),
           _c_spec(), _c_spec()],
        out_specs=[_p_spec()] * 3,
        out_shape=[jax.ShapeDtypeStruct((PR, 128), _f32)] * 3,
    )(s0, s1, glo, ghi, dvp, xp, wb, cst, cst2)


def kernel(x, t, y, fc1_W, fc1_b, fc2_W, fc2_b, conv1_W, conv1_b,
           conv2_W, conv2_b, conv3_W, conv3_b, fc3_W, fc3_b,
           fc4_W, fc4_b, edge_index):
    pad_i = jnp.arange(EPAD - E, dtype=jnp.int32)
    src2d = jnp.concatenate([edge_index[0], N + pad_i % NZSPREAD]
                            ).reshape(EROWS, 128)
    dst2d = jnp.concatenate([edge_index[1], pad_i % N]).reshape(EROWS, 128)

    zeros_hbm = jnp.zeros((ROWS_PER_SUB, 16), _f32)
    ones_tbl = jnp.zeros((NPAD, 16), _f32).at[:N].set(1.0)

    xp = jnp.zeros((NPAD, 16), _f32).at[:N].set(
        jnp.broadcast_to(x, (N, 16))).reshape(PR, 128)
    y0p = jnp.zeros((NPAD, 16), _f32).at[:N].set(
        jnp.broadcast_to(y[0][:, None], (N, 16))).reshape(PR, 128)

    eye8 = jnp.eye(8, dtype=_f32)

    def bd(m):
        return jnp.kron(eye8, m)

    def bd4(w):
        return jnp.stack([
            jnp.stack([bd(w[:16, :16]), bd(w[:16, 16:])]),
            jnp.stack([bd(w[16:, :16]), bd(w[16:, 16:])])])

    def t16(v):
        return jnp.tile(v, 8)

    def pad32(v, n):
        return jnp.zeros((32,), _f32).at[:n].set(v)

    def rows8(*rows):
        c = jnp.zeros((8, 128), _f32)
        for i, r in enumerate(rows):
            c = c.at[i].set(r)
        return c

    fc2p = jnp.zeros((32, 32), _f32).at[:, :26].set(fc2_W)
    b2p = pad32(fc2_b, 26)
    w1ap = jnp.zeros((32, 32), _f32).at[:26].set(conv1_W[:26])
    wx1 = conv1_W[26:29].sum(0)
    tbt = t[:, None] * conv1_W[29:32].sum(0)[None, :]
    w3p = jnp.zeros((32, 32), _f32).at[:, :26].set(conv3_W)
    b3p = pad32(conv3_b, 26)
    fc3p = jnp.zeros((32, 32), _f32).at[:26].set(fc3_W)
    fc4p = jnp.zeros((32, 16), _f32).at[:, 0].set(fc4_W[:, 0])

    fc1r = jnp.broadcast_to(fc1_W[0], (32,))

    wb_init = jnp.concatenate([bd4(fc2p), bd4(w1ap)], 0)
    cst_init = (
        rows8(t16(fc1r[:16]), t16(fc1r[16:]), t16(fc1_b[:16]),
              t16(fc1_b[16:]), t16(b2p[:16]), t16(b2p[16:]),
              t16(wx1[:16]), t16(wx1[16:])),
        rows8(t16(tbt[1, :16]), t16(tbt[1, 16:])),
    )
    wb_c2 = bd4(conv2_W)
    cst_c1 = rows8(t16(conv1_b[:16]), t16(conv1_b[16:]))
    wb_c3 = bd4(w3p)
    cst_c2 = rows8(t16(conv2_b[:16]), t16(conv2_b[16:]))
    f3 = bd4(fc3p)
    w1a4 = bd4(w1ap)
    wb_tail = jnp.stack([
        f3[0, 0], f3[1, 0], f3[0, 1], f3[1, 1],
        bd(fc4p[:16]), bd(fc4p[16:]),
        w1a4[0, 0], w1a4[1, 0], w1a4[0, 1], w1a4[1, 1]])
    cst_tail = rows8(t16(b3p[:16]), t16(b3p[16:]), t16(fc3_b[:16]),
                     t16(fc3_b[16:]), jnp.full((128,), fc4_b[0], _f32),
                     jnp.zeros((128,), _f32), t16(wx1[:16]), t16(wx1[16:]))

    dg0, _ = _sc_seg(src2d, dst2d, zeros_hbm, ones_tbl, ones_tbl)
    dvp = _tc_prep(dg0.reshape(PR, 128))

    glo, ghi = _tc_init(y0p, xp, dvp, wb_init, cst_init)
    yrows = [y[0]]
    for i in range(1, T):
        s0, s1 = _seg(src2d, dst2d, zeros_hbm, glo, ghi)
        glo, ghi = _tc_mid(s0, s1, glo, ghi, dvp, wb_c2, cst_c1)
        s0, s1 = _seg(src2d, dst2d, zeros_hbm, glo, ghi)
        glo, ghi = _tc_mid(s0, s1, glo, ghi, dvp, wb_c3, cst_c2)
        s0, s1 = _seg(src2d, dst2d, zeros_hbm, glo, ghi)
        tb_next = tbt[i + 1] if i + 1 < T else jnp.zeros((32,), _f32)
        cst2 = rows8(t16(tb_next[:16]), t16(tb_next[16:]))
        yp, glo, ghi = _tc_tail(s0, s1, glo, ghi, dvp, xp,
                                wb_tail, cst_tail, cst2)
        yrows.append(yp.reshape(NPAD, 16)[:N, 0])

    res = jnp.stack(yrows, 0)
    return res.reshape(-1)
